# baseline (speedup 1.0000x reference)
import functools

import jax
import jax.numpy as jnp
from jax import lax
from jax.experimental import pallas as pl
from jax.experimental.pallas import tpu as pltpu
from jax.experimental.pallas import tpu_sc as plsc

_LANES = 16
_NUM_CORES = 2
_NUM_SUBCORES = 16
_NUM_WORKERS = _NUM_CORES * _NUM_SUBCORES
_STREAMS = 4


def _lookup_call(bids_flat, logits_flat, num_ids, vocab):
    n = bids_flat.shape[0]
    per_worker = n // _NUM_WORKERS
    n_idx_vecs = per_worker // _LANES
    main_rows = _NUM_SUBCORES * _LANES
    extra_rows = range(main_rows, num_ids)
    tbl_len = main_rows + _LANES * len(extra_rows)
    seg = vocab // _STREAMS
    assert seg * _STREAMS == vocab and per_worker % _LANES == 0
    n_chunks = (vocab + _LANES - 1) // _LANES
    chunk_pad = n_chunks * _LANES

    mesh = plsc.VectorSubcoreMesh(core_axis_name="c", subcore_axis_name="s",
                                  num_cores=_NUM_CORES,
                                  num_subcores=_NUM_SUBCORES)

    @functools.partial(
        pl.kernel,
        mesh=mesh,
        out_type=jax.ShapeDtypeStruct((n,), jnp.int32),
        compiler_params=pltpu.CompilerParams(needs_layout_passes=False,
                                             skip_device_barrier=True),
        scratch_types=[
            pltpu.VMEM((_LANES * vocab,), jnp.float32),
            pltpu.VMEM((max(1, len(extra_rows)) * chunk_pad,),
                       jnp.float32),
            pltpu.VMEM((per_worker,), jnp.int32),
            pltpu.VMEM((per_worker,), jnp.int32),
            pltpu.VMEM((tbl_len,), jnp.int32),
            pltpu.VMEM((_LANES,), jnp.int32),
            pltpu.VMEM_SHARED((tbl_len,), jnp.int32),
            pltpu.SemaphoreType.DMA,
            pltpu.SemaphoreType.DMA,
        ],
    )
    def body(logits_hbm, bids_hbm, out_hbm,
             rows_v, lrow_v, bids_v, out_v, tbl_v, res_v, tbl_sh, sem, sem2):
        cid = lax.axis_index("c")
        sid = lax.axis_index("s")
        wid = cid * _NUM_SUBCORES + sid
        base = wid * per_worker
        lane = lax.iota(jnp.int32, _LANES)
        neg_inf = jnp.full((_LANES,), -jnp.inf, jnp.float32)
        zero_i = jnp.zeros((_LANES,), jnp.int32)

        bids_cp = pltpu.async_copy(bids_hbm.at[pl.ds(base, per_worker)],
                                   bids_v, sem)
        lrow_cps = []
        for j, row in enumerate(extra_rows):
            lrow_cps.append(pltpu.async_copy(
                logits_hbm.at[pl.ds(row * vocab, vocab)],
                lrow_v.at[pl.ds(j * chunk_pad, vocab)], sem2))
            tail = vocab + jnp.minimum(lane, chunk_pad - vocab - 1)
            plsc.store_scatter(lrow_v, [j * chunk_pad + tail], neg_inf,
                               mask=lane < chunk_pad - vocab)

        pltpu.sync_copy(
            logits_hbm.at[pl.ds(sid * _LANES * vocab, _LANES * vocab)],
            rows_v)
        stream_base = [lane * vocab + k * seg for k in range(_STREAMS)]

        def col_step(c, accs):
            col_vec = jnp.full((_LANES,), c, jnp.int32)
            out = []
            for k in range(_STREAMS):
                best_val, best_idx = accs[k]
                v = plsc.load_gather(rows_v, [stream_base[k] + col_vec])
                gt = v > best_val
                out.append((jnp.where(gt, v, best_val),
                            jnp.where(gt, col_vec, best_idx)))
            return tuple(out)

        init = tuple((neg_inf, zero_i) for _ in range(_STREAMS))
        accs = lax.fori_loop(0, seg, col_step, init, unroll=4)
        best_val, best_idx = accs[0]
        for k in range(1, _STREAMS):
            v_k, i_k = accs[k]
            gt = v_k > best_val
            best_val = jnp.where(gt, v_k, best_val)
            best_idx = jnp.where(gt, i_k + (k * seg), best_idx)
        res_v[...] = best_idx
        pltpu.sync_copy(res_v, tbl_sh.at[pl.ds(sid * _LANES, _LANES)])

        for cp in lrow_cps:
            cp.wait()

        @pl.when(sid == 0)
        def _():
            for j, row in enumerate(extra_rows):
                res_slot = main_rows + j * _LANES
                buf0 = j * chunk_pad

                def chunk_step(c, carry):
                    bv, bi = carry
                    v = lrow_v[pl.ds(buf0 + c * _LANES, _LANES)]
                    gt = v > bv
                    return (jnp.where(gt, v, bv),
                            jnp.where(gt, jnp.full((_LANES,), c, jnp.int32),
                                      bi))

                bv, bi = lax.fori_loop(0, n_chunks, chunk_step,
                                       (neg_inf, zero_i), unroll=4)
                m = jnp.max(bv)
                col = bi * _LANES + lane
                cand = jnp.where(bv == m, col, jnp.full((_LANES,), vocab,
                                                        jnp.int32))
                res_v[...] = jnp.full((_LANES,), jnp.min(cand), jnp.int32)
                pltpu.sync_copy(res_v, tbl_sh.at[pl.ds(res_slot, _LANES)])

        plsc.subcore_barrier()
        pltpu.sync_copy(tbl_sh, tbl_v)

        bids_cp.wait()
        max_row = jnp.full((_LANES,), num_ids - 1, jnp.int32)

        def idx_step(i, _):
            idx = bids_v[pl.ds(i * _LANES, _LANES)]
            idx = jnp.minimum(jnp.maximum(idx, zero_i), max_row)
            out_v[pl.ds(i * _LANES, _LANES)] = plsc.load_gather(tbl_v, [idx])
            return 0

        lax.fori_loop(0, n_idx_vecs, idx_step, 0, unroll=8)
        pltpu.sync_copy(out_v, out_hbm.at[pl.ds(base, per_worker)])

    return body(logits_flat, bids_flat)


def kernel(byte_ids, logits):
    b, l = byte_ids.shape
    num_ids, vocab = logits.shape
    bids_flat = byte_ids.reshape(-1).astype(jnp.int32)
    out = _lookup_call(bids_flat, logits.reshape(-1), num_ids, vocab)
    return out.reshape(b, l)


# speedup vs baseline: 1.0221x; 1.0221x over previous
import functools

import jax
import jax.numpy as jnp
from jax import lax
from jax.experimental import pallas as pl
from jax.experimental.pallas import tpu as pltpu
from jax.experimental.pallas import tpu_sc as plsc

_LANES = 16
_NUM_CORES = 2
_NUM_SUBCORES = 16
_NUM_WORKERS = _NUM_CORES * _NUM_SUBCORES
_STREAMS = 4


def _lookup_call(bids_flat, logits_flat, num_ids, vocab):
    n = bids_flat.shape[0]
    per_worker = n // _NUM_WORKERS
    n_idx_vecs = per_worker // _LANES
    main_rows = _NUM_SUBCORES * _LANES
    extra_rows = range(main_rows, num_ids)
    tbl_len = main_rows + _LANES * len(extra_rows)
    seg = vocab // _STREAMS
    assert seg * _STREAMS == vocab and per_worker % _LANES == 0
    n_chunks = (vocab + _LANES - 1) // _LANES
    chunk_pad = n_chunks * _LANES

    mesh = plsc.VectorSubcoreMesh(core_axis_name="c", subcore_axis_name="s",
                                  num_cores=_NUM_CORES,
                                  num_subcores=_NUM_SUBCORES)

    @functools.partial(
        pl.kernel,
        mesh=mesh,
        out_type=jax.ShapeDtypeStruct((n,), jnp.int32),
        compiler_params=pltpu.CompilerParams(needs_layout_passes=False,
                                             skip_device_barrier=True),
        scratch_types=[
            pltpu.VMEM((_LANES * vocab,), jnp.float32),
            pltpu.VMEM((max(1, len(extra_rows)) * chunk_pad,),
                       jnp.float32),
            pltpu.VMEM((per_worker,), jnp.int32),
            pltpu.VMEM((per_worker,), jnp.int32),
            pltpu.VMEM((tbl_len,), jnp.int32),
            pltpu.VMEM((_LANES,), jnp.int32),
            pltpu.VMEM_SHARED((tbl_len,), jnp.int32),
            pltpu.SemaphoreType.DMA,
            pltpu.SemaphoreType.DMA,
        ],
    )
    def body(logits_hbm, bids_hbm, out_hbm,
             rows_v, lrow_v, bids_v, out_v, tbl_v, res_v, tbl_sh, sem, sem2):
        cid = lax.axis_index("c")
        sid = lax.axis_index("s")
        wid = cid * _NUM_SUBCORES + sid
        base = wid * per_worker
        lane = lax.iota(jnp.int32, _LANES)
        neg_inf = jnp.full((_LANES,), -jnp.inf, jnp.float32)
        zero_i = jnp.zeros((_LANES,), jnp.int32)

        bids_cp = pltpu.async_copy(bids_hbm.at[pl.ds(base, per_worker)],
                                   bids_v, sem)
        pltpu.sync_copy(
            logits_hbm.at[pl.ds(sid * _LANES * vocab, _LANES * vocab)],
            rows_v)
        lrow_cps = []
        for j, row in enumerate(extra_rows):
            lrow_cps.append(pltpu.async_copy(
                logits_hbm.at[pl.ds(row * vocab, vocab)],
                lrow_v.at[pl.ds(j * chunk_pad, vocab)], sem2))
            tail = vocab + jnp.minimum(lane, chunk_pad - vocab - 1)
            plsc.store_scatter(lrow_v, [j * chunk_pad + tail], neg_inf,
                               mask=lane < chunk_pad - vocab)
        stream_base = [lane * vocab + k * seg for k in range(_STREAMS)]

        def col_step(c, accs):
            col_vec = jnp.full((_LANES,), c, jnp.int32)
            out = []
            for k in range(_STREAMS):
                best_val, best_idx = accs[k]
                v = plsc.load_gather(rows_v, [stream_base[k] + col_vec])
                gt = v > best_val
                out.append((jnp.where(gt, v, best_val),
                            jnp.where(gt, col_vec, best_idx)))
            return tuple(out)

        init = tuple((neg_inf, zero_i) for _ in range(_STREAMS))
        accs = lax.fori_loop(0, seg, col_step, init, unroll=4)
        best_val, best_idx = accs[0]
        for k in range(1, _STREAMS):
            v_k, i_k = accs[k]
            gt = v_k > best_val
            best_val = jnp.where(gt, v_k, best_val)
            best_idx = jnp.where(gt, i_k + (k * seg), best_idx)
        res_v[...] = best_idx
        pltpu.sync_copy(res_v, tbl_sh.at[pl.ds(sid * _LANES, _LANES)])

        for cp in lrow_cps:
            cp.wait()

        @pl.when(sid == 0)
        def _():
            for j, row in enumerate(extra_rows):
                res_slot = main_rows + j * _LANES
                buf0 = j * chunk_pad

                def chunk_step(c, carry):
                    bv, bi = carry
                    v = lrow_v[pl.ds(buf0 + c * _LANES, _LANES)]
                    gt = v > bv
                    return (jnp.where(gt, v, bv),
                            jnp.where(gt, jnp.full((_LANES,), c, jnp.int32),
                                      bi))

                bv, bi = lax.fori_loop(0, n_chunks, chunk_step,
                                       (neg_inf, zero_i), unroll=4)
                m = jnp.max(bv)
                col = bi * _LANES + lane
                cand = jnp.where(bv == m, col, jnp.full((_LANES,), vocab,
                                                        jnp.int32))
                res_v[...] = jnp.full((_LANES,), jnp.min(cand), jnp.int32)
                pltpu.sync_copy(res_v, tbl_sh.at[pl.ds(res_slot, _LANES)])

        plsc.subcore_barrier()
        pltpu.sync_copy(tbl_sh, tbl_v)

        bids_cp.wait()
        max_row = jnp.full((_LANES,), num_ids - 1, jnp.int32)

        def idx_step(i, _):
            idx = bids_v[pl.ds(i * _LANES, _LANES)]
            idx = jnp.minimum(jnp.maximum(idx, zero_i), max_row)
            out_v[pl.ds(i * _LANES, _LANES)] = plsc.load_gather(tbl_v, [idx])
            return 0

        lax.fori_loop(0, n_idx_vecs, idx_step, 0, unroll=4)
        pltpu.sync_copy(out_v, out_hbm.at[pl.ds(base, per_worker)])

    return body(logits_flat, bids_flat)


def kernel(byte_ids, logits):
    b, l = byte_ids.shape
    num_ids, vocab = logits.shape
    bids_flat = byte_ids.reshape(-1).astype(jnp.int32)
    out = _lookup_call(bids_flat, logits.reshape(-1), num_ids, vocab)
    return out.reshape(b, l)


# speedup vs baseline: 1.0691x; 1.0459x over previous
import functools

import jax
import jax.numpy as jnp
from jax import lax
from jax.experimental import pallas as pl
from jax.experimental.pallas import tpu as pltpu
from jax.experimental.pallas import tpu_sc as plsc

_LANES = 16
_NUM_CORES = 2
_NUM_SUBCORES = 16
_NUM_WORKERS = _NUM_CORES * _NUM_SUBCORES
_STREAMS = 4


def _lookup_call(bids_flat, logits_flat, num_ids, vocab):
    n = bids_flat.shape[0]
    per_worker = n // _NUM_WORKERS
    n_idx_vecs = per_worker // _LANES
    main_rows = _NUM_SUBCORES * _LANES
    extra_rows = range(main_rows, num_ids)
    tbl_len = main_rows + _LANES * len(extra_rows)
    seg = vocab // _STREAMS
    assert seg * _STREAMS == vocab and per_worker % _LANES == 0
    n_chunks = (vocab + _LANES - 1) // _LANES
    chunk_pad = n_chunks * _LANES

    mesh = plsc.VectorSubcoreMesh(core_axis_name="c", subcore_axis_name="s",
                                  num_cores=_NUM_CORES,
                                  num_subcores=_NUM_SUBCORES)

    @functools.partial(
        pl.kernel,
        mesh=mesh,
        out_type=jax.ShapeDtypeStruct((n,), jnp.int32),
        compiler_params=pltpu.CompilerParams(needs_layout_passes=False,
                                             skip_device_barrier=True),
        scratch_types=[
            pltpu.VMEM((_LANES * vocab,), jnp.float32),
            pltpu.VMEM((max(1, len(extra_rows)) * chunk_pad,),
                       jnp.float32),
            pltpu.VMEM((per_worker,), jnp.int32),
            pltpu.VMEM((per_worker,), jnp.int32),
            pltpu.VMEM((tbl_len,), jnp.int32),
            pltpu.VMEM((_LANES,), jnp.int32),
            pltpu.VMEM_SHARED((tbl_len,), jnp.int32),
            pltpu.SemaphoreType.DMA,
            pltpu.SemaphoreType.DMA,
        ],
    )
    def body(logits_hbm, bids_hbm, out_hbm,
             rows_v, lrow_v, bids_v, out_v, tbl_v, res_v, tbl_sh, sem, sem2):
        cid = lax.axis_index("c")
        sid = lax.axis_index("s")
        wid = cid * _NUM_SUBCORES + sid
        base = wid * per_worker
        lane = lax.iota(jnp.int32, _LANES)
        neg_inf = jnp.full((_LANES,), -jnp.inf, jnp.float32)
        zero_i = jnp.zeros((_LANES,), jnp.int32)

        bids_cp = pltpu.async_copy(bids_hbm.at[pl.ds(base, per_worker)],
                                   bids_v, sem)
        pltpu.sync_copy(
            logits_hbm.at[pl.ds(sid * _LANES * vocab, _LANES * vocab)],
            rows_v)
        lrow_cps = []
        for j, row in enumerate(extra_rows):
            lrow_cps.append(pltpu.async_copy(
                logits_hbm.at[pl.ds(row * vocab, vocab)],
                lrow_v.at[pl.ds(j * chunk_pad, vocab)], sem2))
            tail = vocab + jnp.minimum(lane, chunk_pad - vocab - 1)
            plsc.store_scatter(lrow_v, [j * chunk_pad + tail], neg_inf,
                               mask=lane < chunk_pad - vocab)
        stream_base = [lane * vocab + k * seg for k in range(_STREAMS)]

        def col_step(c, accs):
            col_vec = jnp.full((_LANES,), c, jnp.int32)
            out = []
            for k in range(_STREAMS):
                best_val, best_idx = accs[k]
                v = plsc.load_gather(rows_v, [stream_base[k] + col_vec])
                gt = v > best_val
                out.append((jnp.where(gt, v, best_val),
                            jnp.where(gt, col_vec, best_idx)))
            return tuple(out)

        init = tuple((neg_inf, zero_i) for _ in range(_STREAMS))
        accs = plsc.parallel_loop(0, seg, 1, unroll=4, carry=init)(col_step)
        best_val, best_idx = accs[0]
        for k in range(1, _STREAMS):
            v_k, i_k = accs[k]
            gt = v_k > best_val
            best_val = jnp.where(gt, v_k, best_val)
            best_idx = jnp.where(gt, i_k + (k * seg), best_idx)
        res_v[...] = best_idx
        pltpu.sync_copy(res_v, tbl_sh.at[pl.ds(sid * _LANES, _LANES)])

        for cp in lrow_cps:
            cp.wait()

        @pl.when(sid == 0)
        def _():
            for j, row in enumerate(extra_rows):
                res_slot = main_rows + j * _LANES
                buf0 = j * chunk_pad

                def chunk_step(c, carry):
                    bv, bi = carry
                    v = lrow_v[pl.ds(buf0 + c * _LANES, _LANES)]
                    gt = v > bv
                    return (jnp.where(gt, v, bv),
                            jnp.where(gt, jnp.full((_LANES,), c, jnp.int32),
                                      bi))

                bv, bi = plsc.parallel_loop(0, n_chunks, 1, unroll=4,
                                            carry=(neg_inf, zero_i))(
                                                chunk_step)
                m = jnp.max(bv)
                col = bi * _LANES + lane
                cand = jnp.where(bv == m, col, jnp.full((_LANES,), vocab,
                                                        jnp.int32))
                res_v[...] = jnp.full((_LANES,), jnp.min(cand), jnp.int32)
                pltpu.sync_copy(res_v, tbl_sh.at[pl.ds(res_slot, _LANES)])

        plsc.subcore_barrier()
        pltpu.sync_copy(tbl_sh, tbl_v)

        bids_cp.wait()
        max_row = jnp.full((_LANES,), num_ids - 1, jnp.int32)

        @plsc.parallel_loop(0, n_idx_vecs, 1, unroll=4)
        def _(i):
            idx = bids_v[pl.ds(i * _LANES, _LANES)]
            idx = jnp.minimum(jnp.maximum(idx, zero_i), max_row)
            out_v[pl.ds(i * _LANES, _LANES)] = plsc.load_gather(tbl_v, [idx])
        pltpu.sync_copy(out_v, out_hbm.at[pl.ds(base, per_worker)])

    return body(logits_flat, bids_flat)


def kernel(byte_ids, logits):
    b, l = byte_ids.shape
    num_ids, vocab = logits.shape
    bids_flat = byte_ids.reshape(-1).astype(jnp.int32)
    out = _lookup_call(bids_flat, logits.reshape(-1), num_ids, vocab)
    return out.reshape(b, l)
